# baseline (speedup 1.0000x reference)
import functools

import jax
import jax.numpy as jnp
from jax import lax
from jax.experimental import pallas as pl
from jax.experimental.pallas import tpu as pltpu
from jax.experimental.pallas import tpu_sc as plsc

_NC = 2
_NS = 16
_NW = _NC * _NS
_L = 16
_C = 64


def _score_kernel(B, D, b_per_w, n_chunks):
    mesh = plsc.VectorSubcoreMesh(core_axis_name="c", subcore_axis_name="s")

    @functools.partial(
        pl.kernel,
        out_type=jax.ShapeDtypeStruct((B,), jnp.float32),
        mesh=mesh,
        compiler_params=pltpu.CompilerParams(
            needs_layout_passes=False, use_tc_tiling_on_sc=True),
        scratch_types=[
            pltpu.VMEM((b_per_w,), jnp.int32),
            pltpu.VMEM((b_per_w,), jnp.int32),
            pltpu.VMEM((b_per_w,), jnp.int32),
            pltpu.VMEM((_C, 64), jnp.float32),
            pltpu.VMEM((_C, 64), jnp.float32),
            pltpu.VMEM((_C, 64), jnp.float32),
            pltpu.VMEM((_C, 64), jnp.float32),
            pltpu.VMEM((_C, 64), jnp.float32),
            pltpu.VMEM((_C, 64), jnp.float32),
            pltpu.VMEM((_C * _L,), jnp.float32),
            pltpu.VMEM((b_per_w,), jnp.float32),
            pltpu.SemaphoreType.DMA,
        ],
    )
    def k(e1_hbm, rel_hbm, e2_hbm, er_hbm, ei_hbm, rr_hbm, ri_hbm, out_hbm,
          e1_v, rel_v, e2_v, e1r, e1i, e2r, e2i, wr, wi, part,
          score_v, sem):
        wid = lax.axis_index("s") * _NC + lax.axis_index("c")
        base = wid * b_per_w
        pltpu.sync_copy(e1_hbm.at[pl.ds(base, b_per_w)], e1_v)
        pltpu.sync_copy(rel_hbm.at[pl.ds(base, b_per_w)], rel_v)
        pltpu.sync_copy(e2_hbm.at[pl.ds(base, b_per_w)], e2_v)

        @pl.loop(0, n_chunks)
        def chunk_loop(c):
            off = c * _C

            @pl.loop(0, _C // _L)
            def fetch_loop(g):
                v1 = e1_v[pl.ds(off + g * _L, _L)]
                v2 = e2_v[pl.ds(off + g * _L, _L)]
                vw = rel_v[pl.ds(off + g * _L, _L)]
                for j in range(_L):
                    r = g * _L + j
                    i1h, i1l = v1[j] >> 3, v1[j] & 7
                    i2h, i2l = v2[j] >> 3, v2[j] & 7
                    iwh, iwl = vw[j] >> 3, vw[j] & 7
                    pltpu.async_copy(er_hbm.at[v1[j]], e1r.at[r], sem)
                    pltpu.async_copy(ei_hbm.at[i1h, i1l], e1i.at[r], sem)
                    pltpu.async_copy(er_hbm.at[v2[j]], e2r.at[r], sem)
                    pltpu.async_copy(ei_hbm.at[i2h, i2l], e2i.at[r], sem)
                    pltpu.async_copy(rr_hbm.at[iwh, iwl], wr.at[r], sem)
                    pltpu.async_copy(ri_hbm.at[iwh, iwl], wi.at[r], sem)

            for buf in (e1r, e1i, e2r, e2i, wr, wi):
                pltpu.make_async_copy(
                    ri_hbm.at[pl.ds(0, _C // 8)], buf, sem).wait()

            @pl.loop(0, _C)
            def row_loop(r):
                acc = None
                for kk in range(D // _L):
                    sl = pl.ds(kk * _L, _L)
                    a_r = e1r[r, sl]
                    a_i = e1i[r, sl]
                    b_r = e2r[r, sl]
                    b_i = e2i[r, sl]
                    w_r = wr[r, sl]
                    w_i = wi[r, sl]
                    t1 = w_r * a_r - w_i * a_i
                    t2 = w_r * a_i + w_i * a_r
                    term = b_r * t1 + b_i * t2
                    acc = term if acc is None else acc + term
                part[pl.ds(r * _L, _L)] = acc

            @pl.loop(0, _C // _L)
            def red_loop(g):
                rowbase = g * (_L * _L) + lax.iota(jnp.int32, _L) * _L
                s = None
                for col in range(_L):
                    v = plsc.load_gather(part, [rowbase + col])
                    s = v if s is None else s + v
                score_v[pl.ds(off + g * _L, _L)] = s

        pltpu.sync_copy(score_v, out_hbm.at[pl.ds(base, b_per_w)])

    return k


def kernel(e1_idx, rel_idx, e2_idx, emb_e_real, emb_e_img,
           emb_rel_real, emb_rel_img):
    B = e1_idx.shape[0]
    D = emb_e_real.shape[1]
    b_per_w = B // _NW
    n_chunks = b_per_w // _C
    ne = emb_e_real.shape[0]
    nr = emb_rel_real.shape[0]
    er3 = emb_e_real
    ei3 = emb_e_img.reshape(ne // 8, 8, D)
    rr3 = emb_rel_real.reshape(nr // 8, 8, D)
    ri3 = emb_rel_img.reshape(nr // 8, 8, D)
    k = _score_kernel(B, D, b_per_w, n_chunks)
    return k(e1_idx.astype(jnp.int32), rel_idx.astype(jnp.int32),
             e2_idx.astype(jnp.int32), er3, ei3, rr3, ri3)


# speedup vs baseline: 1.0869x; 1.0869x over previous
import functools

import jax
import jax.numpy as jnp
from jax import lax
from jax.experimental import pallas as pl
from jax.experimental.pallas import tpu as pltpu
from jax.experimental.pallas import tpu_sc as plsc

_NC = 2
_NS = 16
_NW = _NC * _NS
_L = 16
_C = 64


def _score_kernel(B, D, b_per_w, n_chunks):
    mesh = plsc.VectorSubcoreMesh(core_axis_name="c", subcore_axis_name="s")

    @functools.partial(
        pl.kernel,
        out_type=jax.ShapeDtypeStruct((B,), jnp.float32),
        mesh=mesh,
        compiler_params=pltpu.CompilerParams(
            needs_layout_passes=False, use_tc_tiling_on_sc=True),
        scratch_types=[
            pltpu.VMEM((b_per_w,), jnp.int32),
            pltpu.VMEM((b_per_w,), jnp.int32),
            pltpu.VMEM((b_per_w,), jnp.int32),
            pltpu.VMEM((_C, 64), jnp.float32),
            pltpu.VMEM((_C, 64), jnp.float32),
            pltpu.VMEM((_C, 64), jnp.float32),
            pltpu.VMEM((_C, 64), jnp.float32),
            pltpu.VMEM((_C, 64), jnp.float32),
            pltpu.VMEM((_C, 64), jnp.float32),
            pltpu.VMEM((_C * _L,), jnp.float32),
            pltpu.VMEM((b_per_w,), jnp.float32),
            pltpu.SemaphoreType.DMA,
        ],
    )
    def k(e1_hbm, rel_hbm, e2_hbm, er_hbm, ei_hbm, rr_hbm, ri_hbm, out_hbm,
          e1_v, rel_v, e2_v, e1r, e1i, e2r, e2i, wr, wi, part,
          score_v, sem):
        wid = lax.axis_index("s") * _NC + lax.axis_index("c")
        base = wid * b_per_w
        pltpu.sync_copy(e1_hbm.at[pl.ds(base, b_per_w)], e1_v)
        pltpu.sync_copy(rel_hbm.at[pl.ds(base, b_per_w)], rel_v)
        pltpu.sync_copy(e2_hbm.at[pl.ds(base, b_per_w)], e2_v)

        @pl.loop(0, n_chunks)
        def chunk_loop(c):
            off = c * _C

            @pl.loop(0, _C // _L)
            def fetch_loop(g):
                v1 = e1_v[pl.ds(off + g * _L, _L)]
                v2 = e2_v[pl.ds(off + g * _L, _L)]
                vw = rel_v[pl.ds(off + g * _L, _L)]
                for j in range(_L):
                    r = g * _L + j
                    i1h, i1l = v1[j] >> 3, v1[j] & 7
                    i2h, i2l = v2[j] >> 3, v2[j] & 7
                    iwh, iwl = vw[j] >> 3, vw[j] & 7
                    pltpu.async_copy(er_hbm.at[i1h, i1l], e1r.at[r], sem)
                    pltpu.async_copy(ei_hbm.at[i1h, i1l], e1i.at[r], sem)
                    pltpu.async_copy(er_hbm.at[i2h, i2l], e2r.at[r], sem)
                    pltpu.async_copy(ei_hbm.at[i2h, i2l], e2i.at[r], sem)
                    pltpu.async_copy(rr_hbm.at[iwh, iwl], wr.at[r], sem)
                    pltpu.async_copy(ri_hbm.at[iwh, iwl], wi.at[r], sem)

            for buf in (e1r, e1i, e2r, e2i, wr, wi):
                pltpu.make_async_copy(
                    ri_hbm.at[pl.ds(0, _C // 8)], buf, sem).wait()

            @pl.loop(0, _C)
            def row_loop(r):
                acc = None
                for kk in range(D // _L):
                    sl = pl.ds(kk * _L, _L)
                    a_r = e1r[r, sl]
                    a_i = e1i[r, sl]
                    b_r = e2r[r, sl]
                    b_i = e2i[r, sl]
                    w_r = wr[r, sl]
                    w_i = wi[r, sl]
                    t1 = w_r * a_r - w_i * a_i
                    t2 = w_r * a_i + w_i * a_r
                    term = b_r * t1 + b_i * t2
                    acc = term if acc is None else acc + term
                part[pl.ds(r * _L, _L)] = acc

            @pl.loop(0, _C // _L)
            def red_loop(g):
                rowbase = g * (_L * _L) + lax.iota(jnp.int32, _L) * _L
                s = None
                for col in range(_L):
                    v = plsc.load_gather(part, [rowbase + col])
                    s = v if s is None else s + v
                score_v[pl.ds(off + g * _L, _L)] = s

        pltpu.sync_copy(score_v, out_hbm.at[pl.ds(base, b_per_w)])

    return k


def kernel(e1_idx, rel_idx, e2_idx, emb_e_real, emb_e_img,
           emb_rel_real, emb_rel_img):
    B = e1_idx.shape[0]
    D = emb_e_real.shape[1]
    b_per_w = B // _NW
    n_chunks = b_per_w // _C
    ne = emb_e_real.shape[0]
    nr = emb_rel_real.shape[0]
    er3 = emb_e_real.reshape(ne // 8, 8, D)
    ei3 = emb_e_img.reshape(ne // 8, 8, D)
    rr3 = emb_rel_real.reshape(nr // 8, 8, D)
    ri3 = emb_rel_img.reshape(nr // 8, 8, D)
    k = _score_kernel(B, D, b_per_w, n_chunks)
    return k(e1_idx.astype(jnp.int32), rel_idx.astype(jnp.int32),
             e2_idx.astype(jnp.int32), er3, ei3, rr3, ri3)


# speedup vs baseline: 1.1035x; 1.0153x over previous
import functools

import jax
import jax.numpy as jnp
from jax import lax
from jax.experimental import pallas as pl
from jax.experimental.pallas import tpu as pltpu
from jax.experimental.pallas import tpu_sc as plsc

_NC = 2
_NS = 16
_NW = _NC * _NS
_L = 16
_C = 64


def _score_kernel(B, D, b_per_w, n_chunks):
    mesh = plsc.VectorSubcoreMesh(core_axis_name="c", subcore_axis_name="s")

    row_bufs = [pltpu.VMEM((_C, 64), jnp.float32) for _ in range(12)]

    @functools.partial(
        pl.kernel,
        out_type=jax.ShapeDtypeStruct((B,), jnp.float32),
        mesh=mesh,
        compiler_params=pltpu.CompilerParams(
            needs_layout_passes=False, use_tc_tiling_on_sc=True),
        scratch_types=[
            pltpu.VMEM((b_per_w,), jnp.int32),
            pltpu.VMEM((b_per_w,), jnp.int32),
            pltpu.VMEM((b_per_w,), jnp.int32),
            *row_bufs,
            pltpu.VMEM((_C * _L,), jnp.float32),
            pltpu.VMEM((b_per_w,), jnp.float32),
            pltpu.SemaphoreType.DMA,
            pltpu.SemaphoreType.DMA,
        ],
    )
    def k(e1_hbm, rel_hbm, e2_hbm, er_hbm, ei_hbm, rr_hbm, ri_hbm, out_hbm,
          e1_v, rel_v, e2_v, *bufs_and_rest):
        bufs0 = bufs_and_rest[0:6]
        bufs1 = bufs_and_rest[6:12]
        part, score_v, sem0, sem1 = bufs_and_rest[12:16]
        wid = lax.axis_index("s") * _NC + lax.axis_index("c")
        base = wid * b_per_w
        pltpu.sync_copy(e1_hbm.at[pl.ds(base, b_per_w)], e1_v)
        pltpu.sync_copy(rel_hbm.at[pl.ds(base, b_per_w)], rel_v)
        pltpu.sync_copy(e2_hbm.at[pl.ds(base, b_per_w)], e2_v)

        def issue(off, bufs, sem):
            e1r, e1i, e2r, e2i, wr, wi = bufs

            @pl.loop(0, _C // _L)
            def fetch_loop(g):
                v1 = e1_v[pl.ds(off + g * _L, _L)]
                v2 = e2_v[pl.ds(off + g * _L, _L)]
                vw = rel_v[pl.ds(off + g * _L, _L)]
                for j in range(_L):
                    r = g * _L + j
                    i1h, i1l = v1[j] >> 3, v1[j] & 7
                    i2h, i2l = v2[j] >> 3, v2[j] & 7
                    iwh, iwl = vw[j] >> 3, vw[j] & 7
                    pltpu.async_copy(er_hbm.at[i1h, i1l], e1r.at[r], sem)
                    pltpu.async_copy(ei_hbm.at[i1h, i1l], e1i.at[r], sem)
                    pltpu.async_copy(er_hbm.at[i2h, i2l], e2r.at[r], sem)
                    pltpu.async_copy(ei_hbm.at[i2h, i2l], e2i.at[r], sem)
                    pltpu.async_copy(rr_hbm.at[iwh, iwl], wr.at[r], sem)
                    pltpu.async_copy(ri_hbm.at[iwh, iwl], wi.at[r], sem)

        def drain(bufs, sem):
            for buf in bufs:
                pltpu.make_async_copy(
                    ri_hbm.at[pl.ds(0, _C // 8)], buf, sem).wait()

        def compute(off, bufs):
            e1r, e1i, e2r, e2i, wr, wi = bufs

            @pl.loop(0, _C)
            def row_loop(r):
                acc = None
                for kk in range(D // _L):
                    sl = pl.ds(kk * _L, _L)
                    a_r = e1r[r, sl]
                    a_i = e1i[r, sl]
                    b_r = e2r[r, sl]
                    b_i = e2i[r, sl]
                    w_r = wr[r, sl]
                    w_i = wi[r, sl]
                    t1 = w_r * a_r - w_i * a_i
                    t2 = w_r * a_i + w_i * a_r
                    term = b_r * t1 + b_i * t2
                    acc = term if acc is None else acc + term
                part[pl.ds(r * _L, _L)] = acc

            @pl.loop(0, _C // _L)
            def red_loop(g):
                rowbase = g * (_L * _L) + lax.iota(jnp.int32, _L) * _L
                s = None
                for col in range(_L):
                    v = plsc.load_gather(part, [rowbase + col])
                    s = v if s is None else s + v
                score_v[pl.ds(off + g * _L, _L)] = s

        issue(0, bufs0, sem0)

        @pl.loop(0, n_chunks // 2)
        def cc_loop(cc):
            c0 = cc * 2
            issue((c0 + 1) * _C, bufs1, sem1)
            drain(bufs0, sem0)
            compute(c0 * _C, bufs0)

            @pl.when(c0 + 2 < n_chunks)
            def _():
                issue((c0 + 2) * _C, bufs0, sem0)

            drain(bufs1, sem1)
            compute((c0 + 1) * _C, bufs1)

        pltpu.sync_copy(score_v, out_hbm.at[pl.ds(base, b_per_w)])

    return k


def kernel(e1_idx, rel_idx, e2_idx, emb_e_real, emb_e_img,
           emb_rel_real, emb_rel_img):
    B = e1_idx.shape[0]
    ne, D = emb_e_real.shape
    nr = emb_rel_real.shape[0]
    b_per_w = B // _NW
    n_chunks = b_per_w // _C
    er3 = emb_e_real.reshape(ne // 8, 8, D)
    ei3 = emb_e_img.reshape(ne // 8, 8, D)
    rr3 = emb_rel_real.reshape(nr // 8, 8, D)
    ri3 = emb_rel_img.reshape(nr // 8, 8, D)
    k = _score_kernel(B, D, b_per_w, n_chunks)
    return k(e1_idx.astype(jnp.int32), rel_idx.astype(jnp.int32),
             e2_idx.astype(jnp.int32), er3, ei3, rr3, ri3)
